# baseline (speedup 1.0000x reference)
import functools

import jax
import jax.numpy as jnp
import numpy as np
from jax import lax
from jax.experimental import pallas as pl
from jax.experimental.pallas import tpu as pltpu
from jax.experimental.pallas import tpu_sc as plsc

_B, _N, _M, _C = 8, 1000, 64, 91
_NP = 1024
_CP = 128
_BIG = 1e9
_BIG2 = 2e9
_MAXI = 2**30
_NCH = _NP // 16
_MCH = _M // 16

_f32 = jnp.float32
_i32 = jnp.int32
_z = np.int32(0)


def _cost_body(pbt_ref, gbp_ref, pc_ref, oh_ref, cost_ref, aval_ref,
               aidx_ref):
    pc = pc_ref[0]
    mx = jnp.max(pc, axis=1, keepdims=True)
    e = jnp.exp(pc - mx)
    s = jnp.sum(e, axis=1, keepdims=True)
    prob = e / s
    oh = oh_ref[0]
    g = lax.dot_general(oh, prob, (((1,), (1,)), ((), ())),
                        preferred_element_type=_f32)
    cost_class = -jnp.concatenate(
        [g, jnp.zeros((_M, _NP - _N), _f32)], axis=1)
    pbt = pbt_ref[0]
    gbp = gbp_ref[0]
    cb = jnp.abs(pbt[0:1, :] - gbp[:, 0:1])
    cb = cb + jnp.abs(pbt[1:2, :] - gbp[:, 1:2])
    cb = cb + jnp.abs(pbt[2:3, :] - gbp[:, 2:3])
    cb = cb + jnp.abs(pbt[3:4, :] - gbp[:, 3:4])
    colio = lax.broadcasted_iota(_i32, (1, _NP), 1)
    pad = jnp.where(colio >= _N, _BIG, _f32(0.0))
    cost = cb + cost_class + pad
    cost_ref[0] = cost
    colio_b = lax.broadcasted_iota(_i32, (_M, _NP), 1)
    mnb = jnp.min(cost, axis=1, keepdims=True)
    idxb = jnp.min(jnp.where(cost == mnb, colio_b, _MAXI),
                   axis=1, keepdims=True)
    rio = lax.broadcasted_iota(_i32, (_M, 1), 0)
    k64 = lax.broadcasted_iota(_i32, (1, _M), 1)
    sel = rio == k64
    aval_ref[0] = jnp.max(jnp.where(sel, mnb, -_BIG2), axis=0,
                          keepdims=True)
    aidx_ref[0] = jnp.max(jnp.where(sel, idxb, -_MAXI), axis=0,
                          keepdims=True)


def _build_cost(pbt, gbp, lt, oh):
    return pl.pallas_call(
        _cost_body,
        grid=(_B,),
        in_specs=[
            pl.BlockSpec((1, 8, _NP), lambda b: (b, _z, _z)),
            pl.BlockSpec((1, _M, _CP), lambda b: (b, _z, _z)),
            pl.BlockSpec((1, _N, _C), lambda b: (b, _z, _z)),
            pl.BlockSpec((1, _M, _C), lambda b: (b, _z, _z)),
        ],
        out_specs=[
            pl.BlockSpec((1, _M, _NP), lambda b: (b, _z, _z)),
            pl.BlockSpec((1, 1, _M), lambda b: (b, _z, _z)),
            pl.BlockSpec((1, 1, _M), lambda b: (b, _z, _z)),
        ],
        out_shape=[
            jax.ShapeDtypeStruct((_B, _M, _NP), _f32),
            jax.ShapeDtypeStruct((_B, 1, _M), _f32),
            jax.ShapeDtypeStruct((_B, 1, _M), _i32),
        ],
    )(pbt, gbp, lt, oh)


def _sc_solver_body(cost_hbm, aval_hbm, aidx_hbm, rows_hbm, cols_hbm,
                    cost_v, u_v, v_v, minv_v, way_v, uc_v, ur_v, p_v,
                    rdone_v, rows_v, cols_v, aval_v, aidx_v, dma_sem):
    cid = lax.axis_index("c")
    w = lax.axis_index("s")
    iota16 = lax.broadcasted_iota(_i32, (16,), 0)

    def read_i(ref, idx, fill):
        base = (idx // 16) * 16
        ch = ref[pl.ds(base, 16)]
        return jnp.max(jnp.where(iota16 == idx % 16, ch, fill))

    def read_f(ref, idx):
        base = (idx // 16) * 16
        ch = ref[pl.ds(base, 16)]
        return jnp.max(jnp.where(iota16 == idx % 16, ch, -_BIG2))

    def write_i(ref, idx, val):
        base = (idx // 16) * 16
        ch = ref[pl.ds(base, 16)]
        ref[pl.ds(base, 16)] = jnp.where(iota16 == idx % 16, val, ch)

    def write_f(ref, idx, val):
        base = (idx // 16) * 16
        ch = ref[pl.ds(base, 16)]
        ref[pl.ds(base, 16)] = jnp.where(iota16 == idx % 16, val, ch)

    def argmin_pass(masked_fn):
        def p1(c, carry1):
            rmin, ridx = carry1
            masked = masked_fn(c)
            upd = masked < rmin
            rmin = jnp.where(upd, masked, rmin)
            ridx = jnp.where(upd, c * 16 + iota16, ridx)
            return (rmin, ridx)

        rmin0 = jnp.full((16,), _BIG2, _f32)
        ridx0 = jnp.full((16,), _MAXI, _i32)
        rmin, ridx = lax.fori_loop(_i32(0), _i32(_NCH), p1, (rmin0, ridx0))
        delta = jnp.min(rmin)
        j1 = jnp.min(jnp.where(rmin == delta, ridx, _MAXI))
        return delta, j1

    @pl.when((cid == 0) & (w < _B))
    def _():
        cost_cp = pltpu.async_copy(cost_hbm.at[w], cost_v, dma_sem)
        pltpu.sync_copy(aval_hbm.at[w], aval_v)
        pltpu.sync_copy(aidx_hbm.at[w], aidx_v)

        def zinit(c, carry):
            for k in range(4):
                sl = pl.ds(c * 64 + k * 16, 16)
                v_v[sl] = jnp.zeros((16,), _f32)
                p_v[sl] = jnp.full((16,), -1, _i32)
            return carry

        lax.fori_loop(_i32(0), _i32(_NCH // 4), zinit, _z)

        def uinit(c, carry):
            sl = pl.ds(c * 16, 16)
            u_v[sl] = jnp.zeros((16,), _f32)
            rdone_v[sl] = jnp.zeros((16,), _i32)
            return carry

        lax.fori_loop(_i32(0), _i32(_MCH), uinit, _z)

        def rowA(g, carry):
            idx_ch = aidx_v[pl.ds(g * 16, 16)]
            val_ch = aval_v[pl.ds(g * 16, 16)]
            for k in range(16):
                i = g * 16 + k
                j1 = idx_ch[k]
                pj1 = read_i(p_v, j1, -_MAXI)

                @pl.when(pj1 == -1)
                def _(j1=j1, i=i, dv=val_ch[k]):
                    write_i(p_v, j1, i)
                    write_f(u_v, i, dv)
                    write_i(rdone_v, i, _i32(1))

            return carry

        lax.fori_loop(_i32(0), _i32(_MCH), rowA, _z)

        cost_cp.wait()

        def rowB(i, carry):
            done_row = read_i(rdone_v, i, -_MAXI)

            @pl.when(done_row == 0)
            def _():
                def sinit(c, carry2):
                    for k in range(4):
                        sl = pl.ds(c * 64 + k * 16, 16)
                        minv_v[sl] = jnp.full((16,), _BIG, _f32)
                        way_v[sl] = jnp.full((16,), -1, _i32)
                        uc_v[sl] = jnp.zeros((16,), _i32)
                    return carry2

                lax.fori_loop(_i32(0), _i32(_NCH // 4), sinit, _z)

                def rinit(c, carry2):
                    sl = pl.ds(c * 16, 16)
                    ur_v[sl] = jnp.zeros((16,), _i32)
                    return carry2

                lax.fori_loop(_i32(0), _i32(_MCH), rinit, _z)

                def sbody(st):
                    i0, j0, _done = st
                    write_i(ur_v, i0, _i32(1))
                    jj = jnp.maximum(j0, _i32(0))
                    basej = (jj // 16) * 16
                    chj = uc_v[pl.ds(basej, 16)]
                    uc_v[pl.ds(basej, 16)] = jnp.where(
                        (iota16 == jj % 16) & (j0 >= 0), 1, chj)
                    ui0 = read_f(u_v, i0)

                    def p1(c, carry1):
                        rmin, ridx = carry1
                        for k in range(4):
                            cc = c * 4 + k
                            sl = pl.ds(cc * 16, 16)
                            free = uc_v[sl] == 0
                            cur = cost_v[i0, sl] - ui0 - v_v[sl]
                            minvc = minv_v[sl]
                            better = free & (cur < minvc)
                            minvc = jnp.where(better, cur, minvc)
                            minv_v[sl] = minvc
                            way_v[sl] = jnp.where(better, j0, way_v[sl])
                            masked = jnp.where(free, minvc, _BIG2)
                            updm = masked < rmin
                            rmin = jnp.where(updm, masked, rmin)
                            ridx = jnp.where(updm, cc * 16 + iota16, ridx)
                        return (rmin, ridx)

                    rmin0 = jnp.full((16,), _BIG2, _f32)
                    ridx0 = jnp.full((16,), _MAXI, _i32)
                    rmin, ridx = lax.fori_loop(_i32(0), _i32(_NCH // 4), p1,
                                               (rmin0, ridx0))
                    delta = jnp.min(rmin)
                    j1 = jnp.min(jnp.where(rmin == delta, ridx, _MAXI))

                    def p2(c, carry3):
                        for k in range(4):
                            sl = pl.ds((c * 4 + k) * 16, 16)
                            freem = uc_v[sl] == 0
                            v_v[sl] = v_v[sl] - jnp.where(
                                freem, _f32(0.0), delta)
                            minv_v[sl] = minv_v[sl] - jnp.where(
                                freem, delta, _f32(0.0))
                        return carry3

                    lax.fori_loop(_i32(0), _i32(_NCH // 4), p2, _z)

                    def p3(c, carry3):
                        sl = pl.ds(c * 16, 16)
                        urc = ur_v[sl]
                        u_v[sl] = u_v[sl] + jnp.where(urc != 0, delta,
                                                      _f32(0.0))
                        return carry3

                    lax.fori_loop(_i32(0), _i32(_MCH), p3, _z)

                    pj1 = read_i(p_v, j1, -_MAXI)
                    done = pj1 == -1
                    i0n = jnp.where(done, i0, pj1)
                    return (i0n, j1, done)

                st = lax.while_loop(lambda st: jnp.logical_not(st[2]),
                                    sbody, (i, _i32(-1), jnp.bool_(False)))
                j0 = st[1]

                def abody(jcur):
                    jprev = read_i(way_v, jcur, -_MAXI)
                    jp = jnp.maximum(jprev, _i32(0))
                    pprev = read_i(p_v, jp, -_MAXI)
                    val = jnp.where(jprev == -1, i, pprev)
                    write_i(p_v, jcur, val)
                    return jprev

                lax.while_loop(lambda j: j != -1, abody, j0)

            return carry

        lax.fori_loop(_i32(0), _i32(_M), rowB, _z)

        def ext(c, base):
            for k in range(4):
                cc = c * 4 + k
                sl = pl.ds(cc * 16, 16)
                pc = p_v[sl]
                mask = pc >= 0
                a = jnp.where(mask, _i32(1), _i32(0))
                incl = plsc.cumsum(a)
                excl = incl - a
                ranks = base + excl
                colvals = cc * 16 + iota16
                plsc.store_scatter(rows_v, [ranks], colvals, mask=mask)
                plsc.store_scatter(cols_v, [ranks], pc, mask=mask)
                base = base + jnp.max(incl)
            return base

        lax.fori_loop(_i32(0), _i32(_NCH // 4), ext, _z)

        pltpu.sync_copy(rows_v, rows_hbm.at[w])
        pltpu.sync_copy(cols_v, cols_hbm.at[w])


_sc_solver = functools.partial(
    pl.kernel,
    out_type=[
        jax.ShapeDtypeStruct((_B, _M), _i32),
        jax.ShapeDtypeStruct((_B, _M), _i32),
    ],
    mesh=plsc.VectorSubcoreMesh(core_axis_name="c", subcore_axis_name="s",
                                num_cores=1),
    scratch_types=[
        pltpu.VMEM((_M, _NP), _f32),
        pltpu.VMEM((_M,), _f32),
        pltpu.VMEM((_NP,), _f32),
        pltpu.VMEM((_NP,), _f32),
        pltpu.VMEM((_NP,), _i32),
        pltpu.VMEM((_NP,), _i32),
        pltpu.VMEM((_M,), _i32),
        pltpu.VMEM((_NP,), _i32),
        pltpu.VMEM((_M,), _i32),
        pltpu.VMEM((_M,), _i32),
        pltpu.VMEM((_M,), _i32),
        pltpu.VMEM((_M,), _f32),
        pltpu.VMEM((_M,), _i32),
        pltpu.SemaphoreType.DMA,
    ],
    compiler_params=pltpu.CompilerParams(needs_layout_passes=False),
)(_sc_solver_body)


def kernel(pred_boxes, pred_obj, pred_class, gt_boxes, gt_labels):
    del pred_obj
    pbt = jnp.zeros((_B, 8, _NP), _f32).at[:, :4, :_N].set(
        pred_boxes.astype(_f32).transpose(0, 2, 1))
    gbp = jnp.zeros((_B, _M, _CP), _f32).at[:, :, :4].set(
        gt_boxes.astype(_f32))
    pc = pred_class.astype(_f32)
    oh = (gt_labels[:, :, None] ==
          jnp.arange(_C, dtype=gt_labels.dtype)[None, None, :]).astype(_f32)

    cost, aval, aidx = _build_cost(pbt, gbp, pc, oh)
    row_ind, col_ind = _sc_solver(cost, aval.reshape(_B, _M),
                                  aidx.reshape(_B, _M))
    return (row_ind, col_ind)


# speedup vs baseline: 1.0019x; 1.0019x over previous
import functools

import jax
import jax.numpy as jnp
import numpy as np
from jax import lax
from jax.experimental import pallas as pl
from jax.experimental.pallas import tpu as pltpu
from jax.experimental.pallas import tpu_sc as plsc

_B, _N, _M, _C = 8, 1000, 64, 91
_NP = 1024
_CP = 128
_BIG = 1e9
_BIG2 = 2e9
_MAXI = 2**30
_NCH = _NP // 16
_MCH = _M // 16

_f32 = jnp.float32
_i32 = jnp.int32
_z = np.int32(0)


def _cost_body(pbt_ref, gbp_ref, pc_ref, oh_ref, cost_ref, aval_ref,
               aidx_ref):
    pcr = pc_ref[0]
    classio = lax.broadcasted_iota(_i32, (1, _CP), 1)
    cmask = classio < _C
    pc = jnp.where(cmask, pcr, -_BIG)
    mx = jnp.max(pc, axis=1, keepdims=True)
    e = jnp.where(cmask, jnp.exp(pc - mx), _f32(0.0))
    s = jnp.sum(e, axis=1, keepdims=True)
    prob = e / s
    oh = oh_ref[0]
    g = lax.dot_general(oh, prob, (((1,), (1,)), ((), ())),
                        preferred_element_type=_f32)
    cost_class = -g
    pbt = pbt_ref[0]
    gbp = gbp_ref[0]
    cb = jnp.abs(pbt[0:1, :] - gbp[:, 0:1])
    cb = cb + jnp.abs(pbt[1:2, :] - gbp[:, 1:2])
    cb = cb + jnp.abs(pbt[2:3, :] - gbp[:, 2:3])
    cb = cb + jnp.abs(pbt[3:4, :] - gbp[:, 3:4])
    colio = lax.broadcasted_iota(_i32, (1, _NP), 1)
    cost = jnp.where(colio >= _N, _BIG, cb + cost_class)
    cost_ref[0] = cost
    colio_b = lax.broadcasted_iota(_i32, (_M, _NP), 1)
    mnb = jnp.min(cost, axis=1, keepdims=True)
    idxb = jnp.min(jnp.where(cost == mnb, colio_b, _MAXI),
                   axis=1, keepdims=True)
    rio = lax.broadcasted_iota(_i32, (_M, 1), 0)
    k64 = lax.broadcasted_iota(_i32, (1, _M), 1)
    sel = rio == k64
    aval_ref[0] = jnp.max(jnp.where(sel, mnb, -_BIG2), axis=0,
                          keepdims=True)
    aidx_ref[0] = jnp.max(jnp.where(sel, idxb, -_MAXI), axis=0,
                          keepdims=True)


def _build_cost(pbt, gbp, lt, oh):
    return pl.pallas_call(
        _cost_body,
        grid=(_B,),
        in_specs=[
            pl.BlockSpec((1, 8, _NP), lambda b: (b, _z, _z)),
            pl.BlockSpec((1, _M, _CP), lambda b: (b, _z, _z)),
            pl.BlockSpec((1, _NP, _CP), lambda b: (b, _z, _z)),
            pl.BlockSpec((1, _M, _CP), lambda b: (b, _z, _z)),
        ],
        out_specs=[
            pl.BlockSpec((1, _M, _NP), lambda b: (b, _z, _z)),
            pl.BlockSpec((1, 1, _M), lambda b: (b, _z, _z)),
            pl.BlockSpec((1, 1, _M), lambda b: (b, _z, _z)),
        ],
        out_shape=[
            jax.ShapeDtypeStruct((_B, _M, _NP), _f32),
            jax.ShapeDtypeStruct((_B, 1, _M), _f32),
            jax.ShapeDtypeStruct((_B, 1, _M), _i32),
        ],
    )(pbt, gbp, lt, oh)


def _sc_solver_body(cost_hbm, aval_hbm, aidx_hbm, rows_hbm, cols_hbm,
                    cost_v, u_v, v_v, minv_v, way_v, uc_v, ur_v, p_v,
                    rdone_v, rows_v, cols_v, aval_v, aidx_v, dma_sem):
    cid = lax.axis_index("c")
    w = lax.axis_index("s")
    iota16 = lax.broadcasted_iota(_i32, (16,), 0)

    def read_i(ref, idx, fill):
        base = (idx // 16) * 16
        ch = ref[pl.ds(base, 16)]
        return jnp.max(jnp.where(iota16 == idx % 16, ch, fill))

    def read_f(ref, idx):
        base = (idx // 16) * 16
        ch = ref[pl.ds(base, 16)]
        return jnp.max(jnp.where(iota16 == idx % 16, ch, -_BIG2))

    def write_i(ref, idx, val):
        base = (idx // 16) * 16
        ch = ref[pl.ds(base, 16)]
        ref[pl.ds(base, 16)] = jnp.where(iota16 == idx % 16, val, ch)

    def write_f(ref, idx, val):
        base = (idx // 16) * 16
        ch = ref[pl.ds(base, 16)]
        ref[pl.ds(base, 16)] = jnp.where(iota16 == idx % 16, val, ch)

    def argmin_pass(masked_fn):
        def p1(c, carry1):
            rmin, ridx = carry1
            masked = masked_fn(c)
            upd = masked < rmin
            rmin = jnp.where(upd, masked, rmin)
            ridx = jnp.where(upd, c * 16 + iota16, ridx)
            return (rmin, ridx)

        rmin0 = jnp.full((16,), _BIG2, _f32)
        ridx0 = jnp.full((16,), _MAXI, _i32)
        rmin, ridx = lax.fori_loop(_i32(0), _i32(_NCH), p1, (rmin0, ridx0))
        delta = jnp.min(rmin)
        j1 = jnp.min(jnp.where(rmin == delta, ridx, _MAXI))
        return delta, j1

    @pl.when((cid == 0) & (w < _B))
    def _():
        cost_cp = pltpu.async_copy(cost_hbm.at[w], cost_v, dma_sem)
        pltpu.sync_copy(aval_hbm.at[w], aval_v)
        pltpu.sync_copy(aidx_hbm.at[w], aidx_v)

        def zinit(c, carry):
            for k in range(4):
                sl = pl.ds(c * 64 + k * 16, 16)
                v_v[sl] = jnp.zeros((16,), _f32)
                p_v[sl] = jnp.full((16,), -1, _i32)
            return carry

        lax.fori_loop(_i32(0), _i32(_NCH // 4), zinit, _z)

        def uinit(c, carry):
            sl = pl.ds(c * 16, 16)
            u_v[sl] = jnp.zeros((16,), _f32)
            rdone_v[sl] = jnp.zeros((16,), _i32)
            return carry

        lax.fori_loop(_i32(0), _i32(_MCH), uinit, _z)

        def rowA(g, carry):
            idx_ch = aidx_v[pl.ds(g * 16, 16)]
            val_ch = aval_v[pl.ds(g * 16, 16)]
            for k in range(16):
                i = g * 16 + k
                j1 = idx_ch[k]
                pj1 = read_i(p_v, j1, -_MAXI)

                @pl.when(pj1 == -1)
                def _(j1=j1, i=i, dv=val_ch[k]):
                    write_i(p_v, j1, i)
                    write_f(u_v, i, dv)
                    write_i(rdone_v, i, _i32(1))

            return carry

        lax.fori_loop(_i32(0), _i32(_MCH), rowA, _z)

        cost_cp.wait()

        def rowB(i, carry):
            done_row = read_i(rdone_v, i, -_MAXI)

            @pl.when(done_row == 0)
            def _():
                def sinit(c, carry2):
                    for k in range(4):
                        sl = pl.ds(c * 64 + k * 16, 16)
                        minv_v[sl] = jnp.full((16,), _BIG, _f32)
                        way_v[sl] = jnp.full((16,), -1, _i32)
                        uc_v[sl] = jnp.zeros((16,), _i32)
                    return carry2

                lax.fori_loop(_i32(0), _i32(_NCH // 4), sinit, _z)

                def rinit(c, carry2):
                    sl = pl.ds(c * 16, 16)
                    ur_v[sl] = jnp.zeros((16,), _i32)
                    return carry2

                lax.fori_loop(_i32(0), _i32(_MCH), rinit, _z)

                def sbody(st):
                    i0, j0, _done = st
                    write_i(ur_v, i0, _i32(1))
                    jj = jnp.maximum(j0, _i32(0))
                    basej = (jj // 16) * 16
                    chj = uc_v[pl.ds(basej, 16)]
                    uc_v[pl.ds(basej, 16)] = jnp.where(
                        (iota16 == jj % 16) & (j0 >= 0), 1, chj)
                    ui0 = read_f(u_v, i0)

                    def p1(c, carry1):
                        rmin, ridx = carry1
                        for k in range(4):
                            cc = c * 4 + k
                            sl = pl.ds(cc * 16, 16)
                            free = uc_v[sl] == 0
                            cur = cost_v[i0, sl] - ui0 - v_v[sl]
                            minvc = minv_v[sl]
                            better = free & (cur < minvc)
                            minvc = jnp.where(better, cur, minvc)
                            minv_v[sl] = minvc
                            way_v[sl] = jnp.where(better, j0, way_v[sl])
                            masked = jnp.where(free, minvc, _BIG2)
                            updm = masked < rmin
                            rmin = jnp.where(updm, masked, rmin)
                            ridx = jnp.where(updm, cc * 16 + iota16, ridx)
                        return (rmin, ridx)

                    rmin0 = jnp.full((16,), _BIG2, _f32)
                    ridx0 = jnp.full((16,), _MAXI, _i32)
                    rmin, ridx = lax.fori_loop(_i32(0), _i32(_NCH // 4), p1,
                                               (rmin0, ridx0))
                    delta = jnp.min(rmin)
                    j1 = jnp.min(jnp.where(rmin == delta, ridx, _MAXI))

                    def p2(c, carry3):
                        for k in range(4):
                            sl = pl.ds((c * 4 + k) * 16, 16)
                            freem = uc_v[sl] == 0
                            v_v[sl] = v_v[sl] - jnp.where(
                                freem, _f32(0.0), delta)
                            minv_v[sl] = minv_v[sl] - jnp.where(
                                freem, delta, _f32(0.0))
                        return carry3

                    lax.fori_loop(_i32(0), _i32(_NCH // 4), p2, _z)

                    def p3(c, carry3):
                        sl = pl.ds(c * 16, 16)
                        urc = ur_v[sl]
                        u_v[sl] = u_v[sl] + jnp.where(urc != 0, delta,
                                                      _f32(0.0))
                        return carry3

                    lax.fori_loop(_i32(0), _i32(_MCH), p3, _z)

                    pj1 = read_i(p_v, j1, -_MAXI)
                    done = pj1 == -1
                    i0n = jnp.where(done, i0, pj1)
                    return (i0n, j1, done)

                st = lax.while_loop(lambda st: jnp.logical_not(st[2]),
                                    sbody, (i, _i32(-1), jnp.bool_(False)))
                j0 = st[1]

                def abody(jcur):
                    jprev = read_i(way_v, jcur, -_MAXI)
                    jp = jnp.maximum(jprev, _i32(0))
                    pprev = read_i(p_v, jp, -_MAXI)
                    val = jnp.where(jprev == -1, i, pprev)
                    write_i(p_v, jcur, val)
                    return jprev

                lax.while_loop(lambda j: j != -1, abody, j0)

            return carry

        lax.fori_loop(_i32(0), _i32(_M), rowB, _z)

        def ext(c, base):
            for k in range(4):
                cc = c * 4 + k
                sl = pl.ds(cc * 16, 16)
                pc = p_v[sl]
                mask = pc >= 0
                a = jnp.where(mask, _i32(1), _i32(0))
                incl = plsc.cumsum(a)
                excl = incl - a
                ranks = base + excl
                colvals = cc * 16 + iota16
                plsc.store_scatter(rows_v, [ranks], colvals, mask=mask)
                plsc.store_scatter(cols_v, [ranks], pc, mask=mask)
                base = base + jnp.max(incl)
            return base

        lax.fori_loop(_i32(0), _i32(_NCH // 4), ext, _z)

        pltpu.sync_copy(rows_v, rows_hbm.at[w])
        pltpu.sync_copy(cols_v, cols_hbm.at[w])


_sc_solver = functools.partial(
    pl.kernel,
    out_type=[
        jax.ShapeDtypeStruct((_B, _M), _i32),
        jax.ShapeDtypeStruct((_B, _M), _i32),
    ],
    mesh=plsc.VectorSubcoreMesh(core_axis_name="c", subcore_axis_name="s",
                                num_cores=1),
    scratch_types=[
        pltpu.VMEM((_M, _NP), _f32),
        pltpu.VMEM((_M,), _f32),
        pltpu.VMEM((_NP,), _f32),
        pltpu.VMEM((_NP,), _f32),
        pltpu.VMEM((_NP,), _i32),
        pltpu.VMEM((_NP,), _i32),
        pltpu.VMEM((_M,), _i32),
        pltpu.VMEM((_NP,), _i32),
        pltpu.VMEM((_M,), _i32),
        pltpu.VMEM((_M,), _i32),
        pltpu.VMEM((_M,), _i32),
        pltpu.VMEM((_M,), _f32),
        pltpu.VMEM((_M,), _i32),
        pltpu.SemaphoreType.DMA,
    ],
    compiler_params=pltpu.CompilerParams(needs_layout_passes=False),
)(_sc_solver_body)


def kernel(pred_boxes, pred_obj, pred_class, gt_boxes, gt_labels):
    del pred_obj
    pbt = jnp.zeros((_B, 8, _NP), _f32).at[:, :4, :_N].set(
        pred_boxes.astype(_f32).transpose(0, 2, 1))
    gbp = jnp.zeros((_B, _M, _CP), _f32).at[:, :, :4].set(
        gt_boxes.astype(_f32))
    pc = pred_class.astype(_f32)
    oh = (gt_labels[:, :, None] ==
          jnp.arange(_CP, dtype=gt_labels.dtype)[None, None, :]).astype(_f32)

    cost, aval, aidx = _build_cost(pbt, gbp, pc, oh)
    row_ind, col_ind = _sc_solver(cost, aval.reshape(_B, _M),
                                  aidx.reshape(_B, _M))
    return (row_ind, col_ind)


# speedup vs baseline: 1.0292x; 1.0273x over previous
import functools

import jax
import jax.numpy as jnp
import numpy as np
from jax import lax
from jax.experimental import pallas as pl
from jax.experimental.pallas import tpu as pltpu
from jax.experimental.pallas import tpu_sc as plsc

_B, _N, _M, _C = 8, 1000, 64, 91
_NP = 1024
_CP = 128
_BIG = 1e9
_BIG2 = 2e9
_MAXI = 2**30
_NCH = _NP // 16
_MCH = _M // 16

_f32 = jnp.float32
_i32 = jnp.int32
_z = np.int32(0)


def _cost_body(pbt_ref, gbp_ref, pc_ref, oh_ref, cost_ref, aval_ref,
               aidx_ref):
    pc = pc_ref[0]
    mx = jnp.max(pc, axis=1, keepdims=True)
    e = jnp.exp(pc - mx)
    s = jnp.sum(e, axis=1, keepdims=True)
    prob = e / s
    oh = oh_ref[0]
    g = lax.dot_general(oh, prob, (((1,), (1,)), ((), ())),
                        preferred_element_type=_f32)
    cost_class = -g
    pbt = pbt_ref[0]
    gbp = gbp_ref[0]
    cb = jnp.abs(pbt[0:1, :] - gbp[:, 0:1])
    cb = cb + jnp.abs(pbt[1:2, :] - gbp[:, 1:2])
    cb = cb + jnp.abs(pbt[2:3, :] - gbp[:, 2:3])
    cb = cb + jnp.abs(pbt[3:4, :] - gbp[:, 3:4])
    colio = lax.broadcasted_iota(_i32, (1, _NP), 1)
    pad = jnp.where(colio >= _N, _BIG, _f32(0.0))
    cost = cb + cost_class + pad
    cost_ref[0] = cost
    colio_b = lax.broadcasted_iota(_i32, (_M, _NP), 1)
    mnb = jnp.min(cost, axis=1, keepdims=True)
    idxb = jnp.min(jnp.where(cost == mnb, colio_b, _MAXI),
                   axis=1, keepdims=True)
    rio = lax.broadcasted_iota(_i32, (_M, 1), 0)
    k64 = lax.broadcasted_iota(_i32, (1, _M), 1)
    sel = rio == k64
    aval_ref[0] = jnp.max(jnp.where(sel, mnb, -_BIG2), axis=0,
                          keepdims=True)
    aidx_ref[0] = jnp.max(jnp.where(sel, idxb, -_MAXI), axis=0,
                          keepdims=True)


def _build_cost(pbt, gbp, lt, oh):
    return pl.pallas_call(
        _cost_body,
        grid=(_B,),
        in_specs=[
            pl.BlockSpec((1, 8, _NP), lambda b: (b, _z, _z)),
            pl.BlockSpec((1, _M, _CP), lambda b: (b, _z, _z)),
            pl.BlockSpec((1, _NP, _CP), lambda b: (b, _z, _z)),
            pl.BlockSpec((1, _M, _CP), lambda b: (b, _z, _z)),
        ],
        out_specs=[
            pl.BlockSpec((1, _M, _NP), lambda b: (b, _z, _z)),
            pl.BlockSpec((1, 1, _M), lambda b: (b, _z, _z)),
            pl.BlockSpec((1, 1, _M), lambda b: (b, _z, _z)),
        ],
        out_shape=[
            jax.ShapeDtypeStruct((_B, _M, _NP), _f32),
            jax.ShapeDtypeStruct((_B, 1, _M), _f32),
            jax.ShapeDtypeStruct((_B, 1, _M), _i32),
        ],
    )(pbt, gbp, lt, oh)


def _sc_solver_body(cost_hbm, aval_hbm, aidx_hbm, rows_hbm, cols_hbm,
                    cost_v, u_v, v_v, minv_v, way_v, uc_v, ur_v, p_v,
                    rdone_v, rows_v, cols_v, aval_v, aidx_v, dma_sem,
                    arg_sem):
    cid = lax.axis_index("c")
    w = lax.axis_index("s")
    iota16 = lax.broadcasted_iota(_i32, (16,), 0)

    def read_i(ref, idx, fill):
        base = (idx // 16) * 16
        ch = ref[pl.ds(base, 16)]
        return jnp.max(jnp.where(iota16 == idx % 16, ch, fill))

    def read_f(ref, idx):
        base = (idx // 16) * 16
        ch = ref[pl.ds(base, 16)]
        return jnp.max(jnp.where(iota16 == idx % 16, ch, -_BIG2))

    def write_i(ref, idx, val):
        base = (idx // 16) * 16
        ch = ref[pl.ds(base, 16)]
        ref[pl.ds(base, 16)] = jnp.where(iota16 == idx % 16, val, ch)

    def write_f(ref, idx, val):
        base = (idx // 16) * 16
        ch = ref[pl.ds(base, 16)]
        ref[pl.ds(base, 16)] = jnp.where(iota16 == idx % 16, val, ch)

    def argmin_pass(masked_fn):
        def p1(c, carry1):
            rmin, ridx = carry1
            masked = masked_fn(c)
            upd = masked < rmin
            rmin = jnp.where(upd, masked, rmin)
            ridx = jnp.where(upd, c * 16 + iota16, ridx)
            return (rmin, ridx)

        rmin0 = jnp.full((16,), _BIG2, _f32)
        ridx0 = jnp.full((16,), _MAXI, _i32)
        rmin, ridx = lax.fori_loop(_i32(0), _i32(_NCH), p1, (rmin0, ridx0))
        delta = jnp.min(rmin)
        j1 = jnp.min(jnp.where(rmin == delta, ridx, _MAXI))
        return delta, j1

    @pl.when((cid == 0) & (w < _B))
    def _():
        cost_cp = pltpu.async_copy(cost_hbm.at[w], cost_v, dma_sem)
        aval_cp = pltpu.async_copy(aval_hbm.at[w], aval_v, arg_sem)
        aidx_cp = pltpu.async_copy(aidx_hbm.at[w], aidx_v, arg_sem)

        def zinit(c, carry):
            for k in range(4):
                sl = pl.ds(c * 64 + k * 16, 16)
                v_v[sl] = jnp.zeros((16,), _f32)
                p_v[sl] = jnp.full((16,), -1, _i32)
            return carry

        lax.fori_loop(_i32(0), _i32(_NCH // 4), zinit, _z)

        def uinit(c, carry):
            sl = pl.ds(c * 16, 16)
            u_v[sl] = jnp.zeros((16,), _f32)
            rdone_v[sl] = jnp.zeros((16,), _i32)
            return carry

        lax.fori_loop(_i32(0), _i32(_MCH), uinit, _z)

        aval_cp.wait()
        aidx_cp.wait()

        def rowA(g, carry):
            idx_ch = aidx_v[pl.ds(g * 16, 16)]
            val_ch = aval_v[pl.ds(g * 16, 16)]
            for k in range(16):
                i = g * 16 + k
                j1 = idx_ch[k]
                pj1 = read_i(p_v, j1, -_MAXI)

                @pl.when(pj1 == -1)
                def _(j1=j1, i=i, dv=val_ch[k]):
                    write_i(p_v, j1, i)
                    write_f(u_v, i, dv)
                    write_i(rdone_v, i, _i32(1))

            return carry

        lax.fori_loop(_i32(0), _i32(_MCH), rowA, _z)

        cost_cp.wait()

        def rowB(i, carry):
            done_row = read_i(rdone_v, i, -_MAXI)

            @pl.when(done_row == 0)
            def _():
                def sinit(c, carry2):
                    for k in range(4):
                        sl = pl.ds(c * 64 + k * 16, 16)
                        minv_v[sl] = jnp.full((16,), _BIG, _f32)
                        way_v[sl] = jnp.full((16,), -1, _i32)
                        uc_v[sl] = jnp.zeros((16,), _i32)
                    return carry2

                lax.fori_loop(_i32(0), _i32(_NCH // 4), sinit, _z)

                def rinit(c, carry2):
                    sl = pl.ds(c * 16, 16)
                    ur_v[sl] = jnp.zeros((16,), _i32)
                    return carry2

                lax.fori_loop(_i32(0), _i32(_MCH), rinit, _z)

                def sbody(st):
                    i0, j0, _done = st
                    write_i(ur_v, i0, _i32(1))
                    jj = jnp.maximum(j0, _i32(0))
                    basej = (jj // 16) * 16
                    chj = uc_v[pl.ds(basej, 16)]
                    uc_v[pl.ds(basej, 16)] = jnp.where(
                        (iota16 == jj % 16) & (j0 >= 0), 1, chj)
                    ui0 = read_f(u_v, i0)

                    def p1(c, carry1):
                        rmin, ridx = carry1
                        for k in range(4):
                            cc = c * 4 + k
                            sl = pl.ds(cc * 16, 16)
                            free = uc_v[sl] == 0
                            cur = cost_v[i0, sl] - ui0 - v_v[sl]
                            minvc = minv_v[sl]
                            better = free & (cur < minvc)
                            minvc = jnp.where(better, cur, minvc)
                            minv_v[sl] = minvc
                            way_v[sl] = jnp.where(better, j0, way_v[sl])
                            masked = jnp.where(free, minvc, _BIG2)
                            updm = masked < rmin
                            rmin = jnp.where(updm, masked, rmin)
                            ridx = jnp.where(updm, cc * 16 + iota16, ridx)
                        return (rmin, ridx)

                    rmin0 = jnp.full((16,), _BIG2, _f32)
                    ridx0 = jnp.full((16,), _MAXI, _i32)
                    rmin, ridx = lax.fori_loop(_i32(0), _i32(_NCH // 4), p1,
                                               (rmin0, ridx0))
                    delta = jnp.min(rmin)
                    j1 = jnp.min(jnp.where(rmin == delta, ridx, _MAXI))

                    def p2(c, carry3):
                        for k in range(4):
                            sl = pl.ds((c * 4 + k) * 16, 16)
                            freem = uc_v[sl] == 0
                            v_v[sl] = v_v[sl] - jnp.where(
                                freem, _f32(0.0), delta)
                            minv_v[sl] = minv_v[sl] - jnp.where(
                                freem, delta, _f32(0.0))
                        return carry3

                    lax.fori_loop(_i32(0), _i32(_NCH // 4), p2, _z)

                    def p3(c, carry3):
                        sl = pl.ds(c * 16, 16)
                        urc = ur_v[sl]
                        u_v[sl] = u_v[sl] + jnp.where(urc != 0, delta,
                                                      _f32(0.0))
                        return carry3

                    lax.fori_loop(_i32(0), _i32(_MCH), p3, _z)

                    pj1 = read_i(p_v, j1, -_MAXI)
                    done = pj1 == -1
                    i0n = jnp.where(done, i0, pj1)
                    return (i0n, j1, done)

                st = lax.while_loop(lambda st: jnp.logical_not(st[2]),
                                    sbody, (i, _i32(-1), jnp.bool_(False)))
                j0 = st[1]

                def abody(jcur):
                    jprev = read_i(way_v, jcur, -_MAXI)
                    jp = jnp.maximum(jprev, _i32(0))
                    pprev = read_i(p_v, jp, -_MAXI)
                    val = jnp.where(jprev == -1, i, pprev)
                    write_i(p_v, jcur, val)
                    return jprev

                lax.while_loop(lambda j: j != -1, abody, j0)

            return carry

        lax.fori_loop(_i32(0), _i32(_M), rowB, _z)

        def ext(c, base):
            for k in range(4):
                cc = c * 4 + k
                sl = pl.ds(cc * 16, 16)
                pc = p_v[sl]
                mask = pc >= 0
                a = jnp.where(mask, _i32(1), _i32(0))
                incl = plsc.cumsum(a)
                excl = incl - a
                ranks = base + excl
                colvals = cc * 16 + iota16
                plsc.store_scatter(rows_v, [ranks], colvals, mask=mask)
                plsc.store_scatter(cols_v, [ranks], pc, mask=mask)
                base = base + jnp.max(incl)
            return base

        lax.fori_loop(_i32(0), _i32(_NCH // 4), ext, _z)

        pltpu.sync_copy(rows_v, rows_hbm.at[w])
        pltpu.sync_copy(cols_v, cols_hbm.at[w])


_sc_solver = functools.partial(
    pl.kernel,
    out_type=[
        jax.ShapeDtypeStruct((_B, _M), _i32),
        jax.ShapeDtypeStruct((_B, _M), _i32),
    ],
    mesh=plsc.VectorSubcoreMesh(core_axis_name="c", subcore_axis_name="s",
                                num_cores=1),
    scratch_types=[
        pltpu.VMEM((_M, _NP), _f32),
        pltpu.VMEM((_M,), _f32),
        pltpu.VMEM((_NP,), _f32),
        pltpu.VMEM((_NP,), _f32),
        pltpu.VMEM((_NP,), _i32),
        pltpu.VMEM((_NP,), _i32),
        pltpu.VMEM((_M,), _i32),
        pltpu.VMEM((_NP,), _i32),
        pltpu.VMEM((_M,), _i32),
        pltpu.VMEM((_M,), _i32),
        pltpu.VMEM((_M,), _i32),
        pltpu.VMEM((_M,), _f32),
        pltpu.VMEM((_M,), _i32),
        pltpu.SemaphoreType.DMA,
        pltpu.SemaphoreType.DMA,
    ],
    compiler_params=pltpu.CompilerParams(needs_layout_passes=False),
)(_sc_solver_body)


def kernel(pred_boxes, pred_obj, pred_class, gt_boxes, gt_labels):
    del pred_obj
    pbt = jnp.zeros((_B, 8, _NP), _f32).at[:, :4, :_N].set(
        pred_boxes.astype(_f32).transpose(0, 2, 1))
    gbp = jnp.zeros((_B, _M, _CP), _f32).at[:, :, :4].set(
        gt_boxes.astype(_f32))
    pc = jnp.full((_B, _NP, _CP), -1e30, _f32).at[:, :_N, :_C].set(
        pred_class.astype(_f32))
    oh = (gt_labels[:, :, None] ==
          jnp.arange(_CP, dtype=gt_labels.dtype)[None, None, :]).astype(_f32)

    cost, aval, aidx = _build_cost(pbt, gbp, pc, oh)
    row_ind, col_ind = _sc_solver(cost, aval.reshape(_B, _M),
                                  aidx.reshape(_B, _M))
    return (row_ind, col_ind)
